# baseline (speedup 1.0000x reference)
import functools

import jax
import jax.numpy as jnp
import numpy as np
from jax.experimental import pallas as pl

_EPS = 1e-5
_AVG_D_LOG = float(np.log(17.0))
_N_LAYERS = 4


def _mm_body(x_ref, w_ref, b_ref, o_ref):
    o_ref[...] = (
        jnp.dot(x_ref[...], w_ref[...], preferred_element_type=jnp.float32)
        + b_ref[...]
    )


def _mm(x, w, b, bm=1000):
    m, k = x.shape
    n = w.shape[1]
    assert m % bm == 0, (m, bm)
    return pl.pallas_call(
        _mm_body,
        grid=(m // bm,),
        in_specs=[
            pl.BlockSpec((bm, k), lambda i: (i, 0)),
            pl.BlockSpec((k, n), lambda i: (0, 0)),
            pl.BlockSpec((1, n), lambda i: (0, 0)),
        ],
        out_specs=pl.BlockSpec((bm, n), lambda i: (i, 0)),
        out_shape=jax.ShapeDtypeStruct((m, n), jnp.float32),
    )(x, w, b.reshape(1, n))


def _post_body(h_ref, agg_ref, wh_ref, wabc_ref, b_ref, amp_ref, att_ref,
               sn_ref, o_ref, stat_ref):
    u = jnp.dot(agg_ref[...], wabc_ref[...], preferred_element_type=jnp.float32)
    hp = (
        jnp.dot(h_ref[...], wh_ref[...], preferred_element_type=jnp.float32)
        + u[:, :128]
        + amp_ref[...] * u[:, 128:256]
        + att_ref[...] * u[:, 256:384]
        + b_ref[...]
    )
    hp = hp * sn_ref[...]
    o_ref[...] = hp

    @pl.when(pl.program_id(0) == 0)
    def _():
        stat_ref[...] = jnp.zeros_like(stat_ref)

    stat_ref[0, :] += jnp.sum(hp, axis=0)
    stat_ref[1, :] += jnp.sum(hp * hp, axis=0)


def _post(h, agg, wh, wabc, b, amp, att, sn, bm=1000):
    m = h.shape[0]
    assert m % bm == 0
    return pl.pallas_call(
        _post_body,
        grid=(m // bm,),
        in_specs=[
            pl.BlockSpec((bm, 128), lambda i: (i, 0)),
            pl.BlockSpec((bm, 512), lambda i: (i, 0)),
            pl.BlockSpec((128, 128), lambda i: (0, 0)),
            pl.BlockSpec((512, 384), lambda i: (0, 0)),
            pl.BlockSpec((1, 128), lambda i: (0, 0)),
            pl.BlockSpec((bm, 1), lambda i: (i, 0)),
            pl.BlockSpec((bm, 1), lambda i: (i, 0)),
            pl.BlockSpec((bm, 1), lambda i: (i, 0)),
        ],
        out_specs=[
            pl.BlockSpec((bm, 128), lambda i: (i, 0)),
            pl.BlockSpec((2, 128), lambda i: (0, 0)),
        ],
        out_shape=[
            jax.ShapeDtypeStruct((m, 128), jnp.float32),
            jax.ShapeDtypeStruct((2, 128), jnp.float32),
        ],
    )(h, agg, wh, wabc, b.reshape(1, 128), amp, att, sn)


def _bn_res_body(hin_ref, hp_ref, mu_ref, rs_ref, g_ref, bb_ref, o_ref):
    hp = (hp_ref[...] - mu_ref[...]) * rs_ref[...] * g_ref[...] + bb_ref[...]
    o_ref[...] = hin_ref[...] + jnp.maximum(hp, 0.0)


def _bn_res(h_in, hp, mu, rstd, g, bb, bm=1000):
    m = h_in.shape[0]
    row = lambda a: a.reshape(1, 128)
    return pl.pallas_call(
        _bn_res_body,
        grid=(m // bm,),
        in_specs=[
            pl.BlockSpec((bm, 128), lambda i: (i, 0)),
            pl.BlockSpec((bm, 128), lambda i: (i, 0)),
            pl.BlockSpec((1, 128), lambda i: (0, 0)),
            pl.BlockSpec((1, 128), lambda i: (0, 0)),
            pl.BlockSpec((1, 128), lambda i: (0, 0)),
            pl.BlockSpec((1, 128), lambda i: (0, 0)),
        ],
        out_specs=pl.BlockSpec((bm, 128), lambda i: (i, 0)),
        out_shape=jax.ShapeDtypeStruct((m, 128), jnp.float32),
    )(h_in, hp, row(mu), row(rstd), row(g), row(bb))


def _readout_body(h_ref, w1_ref, b1_ref, w2_ref, b2_ref, w3_ref, b3_ref,
                  o_ref, acc_ref):
    @pl.when(pl.program_id(0) == 0)
    def _():
        acc_ref[...] = jnp.zeros_like(acc_ref)

    acc_ref[0, :] += jnp.sum(h_ref[...], axis=0)

    @pl.when(pl.program_id(0) == pl.num_programs(0) - 1)
    def _():
        hg = acc_ref[...] * (1.0 / h_ref.shape[0] / pl.num_programs(0))
        x = jnp.maximum(jnp.dot(hg, w1_ref[...],
                                preferred_element_type=jnp.float32)
                        + b1_ref[...], 0.0)
        x = jnp.maximum(jnp.dot(x, w2_ref[...],
                                preferred_element_type=jnp.float32)
                        + b2_ref[...], 0.0)
        o_ref[...] = (jnp.dot(x, w3_ref[...],
                              preferred_element_type=jnp.float32)
                      + b3_ref[...])


def _readout(h, r, bm=1000):
    m = h.shape[0]
    return pl.pallas_call(
        _readout_body,
        grid=(m // bm,),
        in_specs=[
            pl.BlockSpec((bm, 128), lambda i: (i, 0)),
            pl.BlockSpec((128, 64), lambda i: (0, 0)),
            pl.BlockSpec((1, 64), lambda i: (0, 0)),
            pl.BlockSpec((64, 32), lambda i: (0, 0)),
            pl.BlockSpec((1, 32), lambda i: (0, 0)),
            pl.BlockSpec((32, 64), lambda i: (0, 0)),
            pl.BlockSpec((1, 64), lambda i: (0, 0)),
        ],
        out_specs=[
            pl.BlockSpec((1, 64), lambda i: (0, 0)),
            pl.BlockSpec((1, 128), lambda i: (0, 0)),
        ],
        out_shape=[
            jax.ShapeDtypeStruct((1, 64), jnp.float32),
            jax.ShapeDtypeStruct((1, 128), jnp.float32),
        ],
    )(h, r['W1'], r['b1'].reshape(1, 64), r['W2'], r['b2'].reshape(1, 32),
      r['W3'], r['b3'].reshape(1, 64))[0]


def _edge_stage(a, bmat, ce_l, src, dst, deg, n):
    m = jax.nn.relu(a[src] + bmat[dst] + ce_l)
    s = jax.ops.segment_sum(m, dst, n)
    s2 = jax.ops.segment_sum(m * m, dst, n)
    mx = jax.ops.segment_max(m, dst, n)
    mx = jnp.where(deg[:, None] > 0, mx, 0.0)
    mn = -jax.ops.segment_max(-m, dst, n)
    mn = jnp.where(deg[:, None] > 0, mn, 0.0)
    return s, s2, mx, mn


def kernel(h, e, snorm_n, snorm_e, params, edge_index):
    src = edge_index[0]
    dst = edge_index[1]
    n = h.shape[0]
    num_e = e.shape[0]

    deg = jnp.zeros((n,), jnp.float32).at[dst].add(1.0)
    degc = jnp.clip(deg, 1.0, None)
    logd = jnp.log(deg + 1.0)
    amp = (logd / _AVG_D_LOG)[:, None]
    att = (_AVG_D_LOG / jnp.clip(logd, _EPS, None))[:, None]

    we_all = jnp.concatenate(
        [params['layers'][l]['pre_W'][256:, :] for l in range(_N_LAYERS)], axis=1)
    be_all = jnp.concatenate(
        [params['layers'][l]['pre_b'] for l in range(_N_LAYERS)], axis=0)
    ce = _mm(e, we_all, be_all)

    h = _mm(h, params['emb_W'], params['emb_b'])

    for l in range(_N_LAYERS):
        p = params['layers'][l]
        h_in = h
        ab = _mm(h, p['pre_W'][:256, :].reshape(2, 128, 128)
                 .transpose(1, 0, 2).reshape(128, 256),
                 jnp.zeros((256,), jnp.float32))
        a, bmat = ab[:, :128], ab[:, 128:]
        s, s2, mx, mn = _edge_stage(
            a, bmat, ce[:, l * 128:(l + 1) * 128], src, dst, deg, n)

        mean = s / degc[:, None]
        var = s2 / degc[:, None] - mean * mean
        std = jnp.sqrt(jax.nn.relu(var) + _EPS)
        agg = jnp.concatenate([mean, mx, mn, std], axis=1)

        wh = p['post_W'][:128, :]
        wabc = jnp.concatenate(
            [p['post_W'][128:640, :], p['post_W'][640:1152, :],
             p['post_W'][1152:1664, :]], axis=1)
        hp, stats = _post(h, agg, wh, wabc, p['post_b'], amp, att, snorm_n)

        mu = stats[0] / n
        var_b = stats[1] / n - mu * mu
        rstd = 1.0 / jnp.sqrt(var_b + _EPS)
        h = _bn_res(h_in, hp, mu, rstd, p['bn_g'], p['bn_b'])

    return _readout(h, params['read'])


# speedup vs baseline: 2.7691x; 2.7691x over previous
import functools

import jax
import jax.numpy as jnp
import numpy as np
from jax import lax
from jax.experimental import pallas as pl
from jax.experimental.pallas import tpu as pltpu
from jax.experimental.pallas import tpu_sc as plsc

_EPS = 1e-5
_AVG_D_LOG = float(np.log(17.0))
_N_LAYERS = 4

_K = 64
_C = 160
_NCHUNK = 64
_NPAD = _C * _NCHUNK


def _mm_body(x_ref, w_ref, b_ref, o_ref):
    o_ref[...] = (
        jnp.dot(x_ref[...], w_ref[...], preferred_element_type=jnp.float32)
        + b_ref[...]
    )


def _mm(x, w, b, bm=1000):
    m, k = x.shape
    n = w.shape[1]
    assert m % bm == 0, (m, bm)
    return pl.pallas_call(
        _mm_body,
        grid=(m // bm,),
        in_specs=[
            pl.BlockSpec((bm, k), lambda i: (i, 0)),
            pl.BlockSpec((k, n), lambda i: (0, 0)),
            pl.BlockSpec((1, n), lambda i: (0, 0)),
        ],
        out_specs=pl.BlockSpec((bm, n), lambda i: (i, 0)),
        out_shape=jax.ShapeDtypeStruct((m, n), jnp.float32),
    )(x, w, b.reshape(1, n))


def _post_body(h_ref, sc_ref, deg_ref, wh_ref, wabc_ref, b_ref,
               sn_ref, o_ref, stat_ref):
    sc = sc_ref[...]
    deg = deg_ref[...]
    degc = jnp.maximum(deg, 1.0)
    mask = deg > 0.0
    mean = sc[:, 0:128] / degc
    var = sc[:, 128:256] / degc - mean * mean
    std = jnp.sqrt(jnp.maximum(var, 0.0) + _EPS)
    mx = jnp.where(mask, sc[:, 256:384], 0.0)
    mn = jnp.where(mask, sc[:, 384:512], 0.0)
    agg = jnp.concatenate([mean, mx, mn, std], axis=1)
    logd = jnp.log(deg + 1.0)
    amp = logd * (1.0 / _AVG_D_LOG)
    att = _AVG_D_LOG / jnp.maximum(logd, _EPS)
    u = jnp.dot(agg, wabc_ref[...], preferred_element_type=jnp.float32)
    hp = (
        jnp.dot(h_ref[...], wh_ref[...], preferred_element_type=jnp.float32)
        + u[:, :128]
        + amp * u[:, 128:256]
        + att * u[:, 256:384]
        + b_ref[...]
    )
    hp = hp * sn_ref[...]
    o_ref[...] = hp

    @pl.when(pl.program_id(0) == 0)
    def _():
        stat_ref[...] = jnp.zeros_like(stat_ref)

    stat_ref[0, :] += jnp.sum(hp, axis=0)
    stat_ref[1, :] += jnp.sum(hp * hp, axis=0)


def _post(h, sc, deg, wh, wabc, b, sn, bm=1000):
    m = h.shape[0]
    assert m % bm == 0
    return pl.pallas_call(
        _post_body,
        grid=(m // bm,),
        in_specs=[
            pl.BlockSpec((bm, 128), lambda i: (i, 0)),
            pl.BlockSpec((bm, 512), lambda i: (i, 0)),
            pl.BlockSpec((bm, 1), lambda i: (i, 0)),
            pl.BlockSpec((128, 128), lambda i: (0, 0)),
            pl.BlockSpec((512, 384), lambda i: (0, 0)),
            pl.BlockSpec((1, 128), lambda i: (0, 0)),
            pl.BlockSpec((bm, 1), lambda i: (i, 0)),
        ],
        out_specs=[
            pl.BlockSpec((bm, 128), lambda i: (i, 0)),
            pl.BlockSpec((2, 128), lambda i: (0, 0)),
        ],
        out_shape=[
            jax.ShapeDtypeStruct((m, 128), jnp.float32),
            jax.ShapeDtypeStruct((2, 128), jnp.float32),
        ],
    )(h, sc, deg, wh, wabc, b.reshape(1, 128), sn)


def _bn_res_body(hin_ref, hp_ref, stat_ref, g_ref, bb_ref, o_ref):
    n = hin_ref.shape[0] * pl.num_programs(0)
    mu = stat_ref[0:1, :] * (1.0 / n)
    var = stat_ref[1:2, :] * (1.0 / n) - mu * mu
    rs = lax.rsqrt(var + _EPS)
    hp = (hp_ref[...] - mu) * rs * g_ref[...] + bb_ref[...]
    o_ref[...] = hin_ref[...] + jnp.maximum(hp, 0.0)


def _bn_res(h_in, hp, stats, g, bb, bm=1000):
    m = h_in.shape[0]
    row = lambda a: a.reshape(1, 128)
    return pl.pallas_call(
        _bn_res_body,
        grid=(m // bm,),
        in_specs=[
            pl.BlockSpec((bm, 128), lambda i: (i, 0)),
            pl.BlockSpec((bm, 128), lambda i: (i, 0)),
            pl.BlockSpec((2, 128), lambda i: (0, 0)),
            pl.BlockSpec((1, 128), lambda i: (0, 0)),
            pl.BlockSpec((1, 128), lambda i: (0, 0)),
        ],
        out_specs=pl.BlockSpec((bm, 128), lambda i: (i, 0)),
        out_shape=jax.ShapeDtypeStruct((m, 128), jnp.float32),
    )(h_in, hp, stats, row(g), row(bb))


def _readout_body(h_ref, w1_ref, b1_ref, w2_ref, b2_ref, w3_ref, b3_ref,
                  o_ref, acc_ref):
    @pl.when(pl.program_id(0) == 0)
    def _():
        acc_ref[...] = jnp.zeros_like(acc_ref)

    acc_ref[0, :] += jnp.sum(h_ref[...], axis=0)

    @pl.when(pl.program_id(0) == pl.num_programs(0) - 1)
    def _():
        hg = acc_ref[...] * (1.0 / h_ref.shape[0] / pl.num_programs(0))
        x = jnp.maximum(jnp.dot(hg, w1_ref[...],
                                preferred_element_type=jnp.float32)
                        + b1_ref[...], 0.0)
        x = jnp.maximum(jnp.dot(x, w2_ref[...],
                                preferred_element_type=jnp.float32)
                        + b2_ref[...], 0.0)
        o_ref[...] = (jnp.dot(x, w3_ref[...],
                              preferred_element_type=jnp.float32)
                      + b3_ref[...])


def _readout(h, r, bm=1000):
    m = h.shape[0]
    return pl.pallas_call(
        _readout_body,
        grid=(m // bm,),
        in_specs=[
            pl.BlockSpec((bm, 128), lambda i: (i, 0)),
            pl.BlockSpec((128, 64), lambda i: (0, 0)),
            pl.BlockSpec((1, 64), lambda i: (0, 0)),
            pl.BlockSpec((64, 32), lambda i: (0, 0)),
            pl.BlockSpec((1, 32), lambda i: (0, 0)),
            pl.BlockSpec((32, 64), lambda i: (0, 0)),
            pl.BlockSpec((1, 64), lambda i: (0, 0)),
        ],
        out_specs=[
            pl.BlockSpec((1, 64), lambda i: (0, 0)),
            pl.BlockSpec((1, 128), lambda i: (0, 0)),
        ],
        out_shape=[
            jax.ShapeDtypeStruct((1, 64), jnp.float32),
            jax.ShapeDtypeStruct((1, 128), jnp.float32),
        ],
    )(h, r['W1'], r['b1'].reshape(1, 64), r['W2'], r['b2'].reshape(1, 32),
      r['W3'], r['b3'].reshape(1, 64))[0]


def _sc_edge_body(a_hbm, b_hbm, ce_hbm, ssrc_hbm, sdst_hbm, perm_hbm,
                  bounds_hbm, out_hbm, deg_hbm,
                  bounds_v, sidx, didx, didxg, pidx, bufa, bufb, bufc,
                  acc, dacc, sem_a, sem_b, sem_c):
    wid = lax.axis_index("s") * 2 + lax.axis_index("c")
    pltpu.sync_copy(bounds_hbm, bounds_v)
    zero16 = jnp.zeros((16,), jnp.float32)
    neg16 = jnp.full((16,), -3e38, jnp.float32)
    pos16 = jnp.full((16,), 3e38, jnp.float32)

    for sub in range(2):
        chunk = wid * 2 + sub
        clo = chunk * _C

        def init_row(i, _):
            for c in range(16):
                acc[i, pl.ds(c * 16, 16)] = zero16
            for c in range(8):
                acc[i, pl.ds(256 + c * 16, 16)] = neg16
                acc[i, pl.ds(384 + c * 16, 16)] = pos16
            return 0

        lax.fori_loop(0, _C + 1, init_row, 0)

        def init_deg(i, _):
            dacc[pl.ds(i * 16, 16)] = zero16
            return 0

        lax.fori_loop(0, (_C + 16) // 16, init_deg, 0)

        bv = bounds_v[pl.ds(chunk, 16)]
        elo = bv[0]
        ehi = bv[1]
        elo8 = (elo // 8) * 8
        nb = (ehi - elo8 + _K - 1) // _K

        nmax = a_hbm.shape[0] - 1

        def batch(t, _):
            base = elo8 + t * _K
            pltpu.sync_copy(ssrc_hbm.at[pl.ds(base, _K)], sidx)
            pltpu.sync_copy(sdst_hbm.at[pl.ds(base, _K)],
                            didx.at[pl.ds(0, _K)])
            pltpu.sync_copy(perm_hbm.at[pl.ds(base, _K)], pidx)
            for g in range(_K // 16):
                didxg[pl.ds(g * 16, 16)] = jnp.minimum(
                    didx[pl.ds(g * 16, 16)], nmax)
            ca = pltpu.async_copy(a_hbm.at[sidx], bufa, sem_a)
            cb = pltpu.async_copy(b_hbm.at[didxg], bufb, sem_b)
            cc = pltpu.async_copy(ce_hbm.at[pidx], bufc, sem_c)
            ca.wait()
            cb.wait()
            cc.wait()

            ones16 = jnp.ones((16,), jnp.float32)
            for g in range(_K // 16):
                dv = didx[pl.ds(g * 16, 16)] - clo
                ok = (dv >= 0) & (dv < _C)
                dvc = jnp.where(ok, dv, _C)
                plsc.addupdate_scatter(dacc, [dvc], ones16, mask=ok)

            def edge(j, _):
                d = didx[pl.ds(j, 16)][0]
                dj = d - clo
                inr = (dj >= 0) & (dj < _C)
                dj = jnp.where(inr, dj, _C)
                for c in range(8):
                    off = c * 16
                    m = jnp.maximum(
                        bufa[j, pl.ds(off, 16)]
                        + bufb[j, pl.ds(off, 16)]
                        + bufc[j, pl.ds(off, 16)], 0.0)
                    plsc.addupdate(acc.at[dj, pl.ds(off, 16)], m)
                    plsc.addupdate(acc.at[dj, pl.ds(128 + off, 16)], m * m)
                    acc[dj, pl.ds(256 + off, 16)] = jnp.maximum(
                        acc[dj, pl.ds(256 + off, 16)], m)
                    acc[dj, pl.ds(384 + off, 16)] = jnp.minimum(
                        acc[dj, pl.ds(384 + off, 16)], m)
                return 0

            lax.fori_loop(0, _K, edge, 0)
            return 0

        lax.fori_loop(0, nb, batch, 0)

        pltpu.sync_copy(acc.at[pl.ds(0, _C)], out_hbm.at[pl.ds(clo, _C)])
        pltpu.sync_copy(dacc.at[pl.ds(0, _C)], deg_hbm.at[pl.ds(clo, _C)])


@functools.partial(jax.jit, static_argnums=())
def _sc_edge(a, bmat, ce_l, ssrc_p, sdst_p, perm_p, bounds):
    mesh = plsc.VectorSubcoreMesh(core_axis_name="c", subcore_axis_name="s",
                                  num_cores=2, num_subcores=16)
    return pl.kernel(
        _sc_edge_body,
        out_type=[
            jax.ShapeDtypeStruct((_NPAD, 512), jnp.float32),
            jax.ShapeDtypeStruct((_NPAD,), jnp.float32),
        ],
        mesh=mesh,
        compiler_params=pltpu.CompilerParams(needs_layout_passes=False),
        scratch_types=[
            pltpu.VMEM((80,), jnp.int32),
            pltpu.VMEM((_K,), jnp.int32),
            pltpu.VMEM((_K + 16,), jnp.int32),
            pltpu.VMEM((_K,), jnp.int32),
            pltpu.VMEM((_K,), jnp.int32),
            pltpu.VMEM((_K, 128), jnp.float32),
            pltpu.VMEM((_K, 128), jnp.float32),
            pltpu.VMEM((_K, 128), jnp.float32),
            pltpu.VMEM((_C + 1, 512), jnp.float32),
            pltpu.VMEM((_C + 16,), jnp.float32),
            pltpu.SemaphoreType.DMA,
            pltpu.SemaphoreType.DMA,
            pltpu.SemaphoreType.DMA,
        ],
    )(a, bmat, ce_l, ssrc_p, sdst_p, perm_p, bounds)


def kernel(h, e, snorm_n, snorm_e, params, edge_index):
    src = edge_index[0]
    dst = edge_index[1]
    n = h.shape[0]

    perm = jnp.argsort(dst).astype(jnp.int32)
    sdst = dst[perm]
    ssrc = src[perm]
    big = jnp.full((_K,), _NPAD, jnp.int32)
    sdst_p = jnp.concatenate([sdst, big])
    ssrc_p = jnp.concatenate([ssrc, jnp.zeros((_K,), jnp.int32)])
    perm_p = jnp.concatenate([perm, jnp.zeros((_K,), jnp.int32)])
    chunk_starts = jnp.arange(0, _NPAD + 1, _C, dtype=jnp.int32)
    bounds = jnp.searchsorted(sdst, chunk_starts).astype(jnp.int32)
    bounds = jnp.concatenate(
        [bounds, jnp.zeros((80 - bounds.shape[0],), jnp.int32)])

    ce_l = [
        _mm(e, params['layers'][l]['pre_W'][256:, :],
            params['layers'][l]['pre_b'])
        for l in range(_N_LAYERS)
    ]

    h = _mm(h, params['emb_W'], params['emb_b'])

    for l in range(_N_LAYERS):
        p = params['layers'][l]
        h_in = h
        a = _mm(h, p['pre_W'][:128, :], jnp.zeros((128,), jnp.float32))
        bmat = _mm(h, p['pre_W'][128:256, :], jnp.zeros((128,), jnp.float32))
        sc_out, deg = _sc_edge(a, bmat, ce_l[l], ssrc_p, sdst_p, perm_p,
                               bounds)

        wh = p['post_W'][:128, :]
        wabc = jnp.concatenate(
            [p['post_W'][128:640, :], p['post_W'][640:1152, :],
             p['post_W'][1152:1664, :]], axis=1)
        hp, stats = _post(h, sc_out, deg.reshape(_NPAD, 1), wh, wabc,
                          p['post_b'], snorm_n)
        h = _bn_res(h_in, hp, stats, p['bn_g'], p['bn_b'])

    return _readout(h, params['read'])


# speedup vs baseline: 3.9513x; 1.4269x over previous
import functools

import jax
import jax.numpy as jnp
import numpy as np
from jax import lax
from jax.experimental import pallas as pl
from jax.experimental.pallas import tpu as pltpu
from jax.experimental.pallas import tpu_sc as plsc

_EPS = 1e-5
_AVG_D_LOG = float(np.log(17.0))
_N_LAYERS = 4

_K = 48
_NSUB = 16
_KB = _K * _NSUB
_C = 160
_NCHUNK = 64
_NPAD = _C * _NCHUNK


def _mm_body(x_ref, w_ref, b_ref, o_ref):
    o_ref[...] = (
        jnp.dot(x_ref[...], w_ref[...], preferred_element_type=jnp.float32)
        + b_ref[...]
    )


def _mm(x, w, b, bm=1000):
    m, k = x.shape
    n = w.shape[1]
    assert m % bm == 0, (m, bm)
    return pl.pallas_call(
        _mm_body,
        grid=(m // bm,),
        in_specs=[
            pl.BlockSpec((bm, k), lambda i: (i, 0)),
            pl.BlockSpec((k, n), lambda i: (0, 0)),
            pl.BlockSpec((1, n), lambda i: (0, 0)),
        ],
        out_specs=pl.BlockSpec((bm, n), lambda i: (i, 0)),
        out_shape=jax.ShapeDtypeStruct((m, n), jnp.float32),
    )(x, w, b.reshape(1, n))


def _post_body(h_ref, sc_ref, deg_ref, wh_ref, wabc_ref, b_ref,
               sn_ref, o_ref, stat_ref):
    sc = sc_ref[...]
    deg = deg_ref[...]
    degc = jnp.maximum(deg, 1.0)
    mask = deg > 0.0
    mean = sc[:, 0:128] / degc
    var = sc[:, 128:256] / degc - mean * mean
    std = jnp.sqrt(jnp.maximum(var, 0.0) + _EPS)
    mx = jnp.where(mask, sc[:, 256:384], 0.0)
    mn = jnp.where(mask, sc[:, 384:512], 0.0)
    agg = jnp.concatenate([mean, mx, mn, std], axis=1)
    logd = jnp.log(deg + 1.0)
    amp = logd * (1.0 / _AVG_D_LOG)
    att = _AVG_D_LOG / jnp.maximum(logd, _EPS)
    u = jnp.dot(agg, wabc_ref[...], preferred_element_type=jnp.float32)
    hp = (
        jnp.dot(h_ref[...], wh_ref[...], preferred_element_type=jnp.float32)
        + u[:, :128]
        + amp * u[:, 128:256]
        + att * u[:, 256:384]
        + b_ref[...]
    )
    hp = hp * sn_ref[...]
    o_ref[...] = hp

    @pl.when(pl.program_id(0) == 0)
    def _():
        stat_ref[...] = jnp.zeros_like(stat_ref)

    stat_ref[0, :] += jnp.sum(hp, axis=0)
    stat_ref[1, :] += jnp.sum(hp * hp, axis=0)


def _post(h, sc, deg, wh, wabc, b, sn, bm=1000):
    m = h.shape[0]
    assert m % bm == 0
    return pl.pallas_call(
        _post_body,
        grid=(m // bm,),
        in_specs=[
            pl.BlockSpec((bm, 128), lambda i: (i, 0)),
            pl.BlockSpec((bm, 512), lambda i: (i, 0)),
            pl.BlockSpec((bm, 1), lambda i: (i, 0)),
            pl.BlockSpec((128, 128), lambda i: (0, 0)),
            pl.BlockSpec((512, 384), lambda i: (0, 0)),
            pl.BlockSpec((1, 128), lambda i: (0, 0)),
            pl.BlockSpec((bm, 1), lambda i: (i, 0)),
        ],
        out_specs=[
            pl.BlockSpec((bm, 128), lambda i: (i, 0)),
            pl.BlockSpec((2, 128), lambda i: (0, 0)),
        ],
        out_shape=[
            jax.ShapeDtypeStruct((m, 128), jnp.float32),
            jax.ShapeDtypeStruct((2, 128), jnp.float32),
        ],
    )(h, sc, deg, wh, wabc, b.reshape(1, 128), sn)


def _bn_res_body(hin_ref, hp_ref, stat_ref, g_ref, bb_ref, o_ref):
    n = hin_ref.shape[0] * pl.num_programs(0)
    mu = stat_ref[0:1, :] * (1.0 / n)
    var = stat_ref[1:2, :] * (1.0 / n) - mu * mu
    rs = lax.rsqrt(var + _EPS)
    hp = (hp_ref[...] - mu) * rs * g_ref[...] + bb_ref[...]
    o_ref[...] = hin_ref[...] + jnp.maximum(hp, 0.0)


def _bn_res(h_in, hp, stats, g, bb, bm=1000):
    m = h_in.shape[0]
    row = lambda a: a.reshape(1, 128)
    return pl.pallas_call(
        _bn_res_body,
        grid=(m // bm,),
        in_specs=[
            pl.BlockSpec((bm, 128), lambda i: (i, 0)),
            pl.BlockSpec((bm, 128), lambda i: (i, 0)),
            pl.BlockSpec((2, 128), lambda i: (0, 0)),
            pl.BlockSpec((1, 128), lambda i: (0, 0)),
            pl.BlockSpec((1, 128), lambda i: (0, 0)),
        ],
        out_specs=pl.BlockSpec((bm, 128), lambda i: (i, 0)),
        out_shape=jax.ShapeDtypeStruct((m, 128), jnp.float32),
    )(h_in, hp, stats, row(g), row(bb))


def _readout_body(h_ref, w1_ref, b1_ref, w2_ref, b2_ref, w3_ref, b3_ref,
                  o_ref, acc_ref):
    @pl.when(pl.program_id(0) == 0)
    def _():
        acc_ref[...] = jnp.zeros_like(acc_ref)

    acc_ref[0, :] += jnp.sum(h_ref[...], axis=0)

    @pl.when(pl.program_id(0) == pl.num_programs(0) - 1)
    def _():
        hg = acc_ref[...] * (1.0 / h_ref.shape[0] / pl.num_programs(0))
        x = jnp.maximum(jnp.dot(hg, w1_ref[...],
                                preferred_element_type=jnp.float32)
                        + b1_ref[...], 0.0)
        x = jnp.maximum(jnp.dot(x, w2_ref[...],
                                preferred_element_type=jnp.float32)
                        + b2_ref[...], 0.0)
        o_ref[...] = (jnp.dot(x, w3_ref[...],
                              preferred_element_type=jnp.float32)
                      + b3_ref[...])


def _readout(h, r, bm=1000):
    m = h.shape[0]
    return pl.pallas_call(
        _readout_body,
        grid=(m // bm,),
        in_specs=[
            pl.BlockSpec((bm, 128), lambda i: (i, 0)),
            pl.BlockSpec((128, 64), lambda i: (0, 0)),
            pl.BlockSpec((1, 64), lambda i: (0, 0)),
            pl.BlockSpec((64, 32), lambda i: (0, 0)),
            pl.BlockSpec((1, 32), lambda i: (0, 0)),
            pl.BlockSpec((32, 64), lambda i: (0, 0)),
            pl.BlockSpec((1, 64), lambda i: (0, 0)),
        ],
        out_specs=[
            pl.BlockSpec((1, 64), lambda i: (0, 0)),
            pl.BlockSpec((1, 128), lambda i: (0, 0)),
        ],
        out_shape=[
            jax.ShapeDtypeStruct((1, 64), jnp.float32),
            jax.ShapeDtypeStruct((1, 128), jnp.float32),
        ],
    )(h, r['W1'], r['b1'].reshape(1, 64), r['W2'], r['b2'].reshape(1, 32),
      r['W3'], r['b3'].reshape(1, 64))[0]


def _sc_edge_body(a_hbm, b_hbm, ce_hbm, ssrc_hbm, sdst_hbm, perm_hbm,
                  bounds_hbm, out_hbm, deg_hbm,
                  bounds_v, sidx, didx, pidx,
                  bufa0, bufb0, bufc0, didxg0,
                  bufa1, bufb1, bufc1, didxg1, acc, dacc,
                  sem_a0, sem_b0, sem_c0, sem_a1, sem_b1, sem_c1):
    wid = lax.axis_index("s") * 2 + lax.axis_index("c")
    pltpu.sync_copy(bounds_hbm, bounds_v)
    zero16 = jnp.zeros((16,), jnp.float32)
    neg16 = jnp.full((16,), -3e38, jnp.float32)
    pos16 = jnp.full((16,), 3e38, jnp.float32)

    for sub in range(2):
        chunk = wid * 2 + sub
        clo = chunk * _C

        def init_row(i, _):
            for c in range(16):
                acc[i, pl.ds(c * 16, 16)] = zero16
            for c in range(8):
                acc[i, pl.ds(256 + c * 16, 16)] = neg16
                acc[i, pl.ds(384 + c * 16, 16)] = pos16
            return 0

        lax.fori_loop(0, _C + 1, init_row, 0)

        def init_deg(i, _):
            dacc[pl.ds(i * 16, 16)] = zero16
            return 0

        lax.fori_loop(0, (_C + 16) // 16, init_deg, 0)

        bv = bounds_v[pl.ds(chunk, 16)]
        elo = bv[0]
        ehi = bv[1]
        elo8 = (elo // 8) * 8

        nmax = a_hbm.shape[0] - 1
        bufs = ((bufa0, bufb0, bufc0, didxg0, sem_a0, sem_b0, sem_c0),
                (bufa1, bufb1, bufc1, didxg1, sem_a1, sem_b1, sem_c1))
        ones16 = jnp.ones((16,), jnp.float32)

        def issue(s, par):
            ba, bb, bc, dg, sa, sb, sc = bufs[par]
            off = s * _K
            for g in range(_K // 16):
                dg[pl.ds(g * 16, 16)] = jnp.minimum(
                    didx[pl.ds(off + g * 16, 16)], nmax)
            pltpu.async_copy(a_hbm.at[sidx.at[pl.ds(off, _K)]], ba, sa)
            pltpu.async_copy(b_hbm.at[dg], bb, sb)
            pltpu.async_copy(ce_hbm.at[pidx.at[pl.ds(off, _K)]], bc, sc)

        def compute(s, par):
            ba, bb, bc, dg, sa, sb, sc = bufs[par]
            pltpu.make_async_copy(a_hbm.at[sidx.at[pl.ds(0, _K)]], ba,
                                  sa).wait()
            pltpu.make_async_copy(b_hbm.at[dg], bb, sb).wait()
            pltpu.make_async_copy(ce_hbm.at[pidx.at[pl.ds(0, _K)]], bc,
                                  sc).wait()
            base = s * _K
            for g in range(_K // 16):
                dv = didx[pl.ds(base + g * 16, 16)] - clo
                ok = (dv >= 0) & (dv < _C)
                dvc = jnp.where(ok, dv, _C)
                plsc.addupdate_scatter(dacc, [dvc], ones16, mask=ok)

            def edge(j, _):
                d = didx[pl.ds(base + j, 16)][0]
                dj = d - clo
                inr = (dj >= 0) & (dj < _C)
                dj = jnp.where(inr, dj, _C)
                for c in range(8):
                    off = c * 16
                    m = jnp.maximum(
                        ba[j, pl.ds(off, 16)]
                        + bb[j, pl.ds(off, 16)]
                        + bc[j, pl.ds(off, 16)], 0.0)
                    plsc.addupdate(acc.at[dj, pl.ds(off, 16)], m)
                    plsc.addupdate(acc.at[dj, pl.ds(128 + off, 16)], m * m)
                    acc[dj, pl.ds(256 + off, 16)] = jnp.maximum(
                        acc[dj, pl.ds(256 + off, 16)], m)
                    acc[dj, pl.ds(384 + off, 16)] = jnp.minimum(
                        acc[dj, pl.ds(384 + off, 16)], m)
                return 0

            lax.fori_loop(0, _K, edge, 0)

        nbig = (ehi - elo8 + _KB - 1) // _KB
        nsub_total = (ehi - elo8 + _K - 1) // _K

        def big_batch(t, _):
            base = elo8 + t * _KB
            pltpu.sync_copy(ssrc_hbm.at[pl.ds(base, _KB)], sidx)
            pltpu.sync_copy(sdst_hbm.at[pl.ds(base, _KB)],
                            didx.at[pl.ds(0, _KB)])
            pltpu.sync_copy(perm_hbm.at[pl.ds(base, _KB)], pidx)
            ns = jnp.minimum(nsub_total - t * _NSUB, _NSUB)
            issue(0, 0)

            def pair(i, _):
                s0 = 2 * i

                @pl.when(s0 + 1 < ns)
                def _():
                    issue(s0 + 1, 1)

                compute(s0, 0)
                s1 = s0 + 1

                @pl.when(s1 < ns)
                def _():
                    @pl.when(s1 + 1 < ns)
                    def _():
                        issue(s1 + 1, 0)

                    compute(s1, 1)

                return 0

            lax.fori_loop(0, (ns + 1) // 2, pair, 0)
            return 0

        lax.fori_loop(0, nbig, big_batch, 0)

        pltpu.sync_copy(acc.at[pl.ds(0, _C)], out_hbm.at[pl.ds(clo, _C)])
        pltpu.sync_copy(dacc.at[pl.ds(0, _C)], deg_hbm.at[pl.ds(clo, _C)])


@functools.partial(jax.jit, static_argnums=())
def _sc_edge(a, bmat, ce_l, ssrc_p, sdst_p, perm_p, bounds):
    mesh = plsc.VectorSubcoreMesh(core_axis_name="c", subcore_axis_name="s",
                                  num_cores=2, num_subcores=16)
    return pl.kernel(
        _sc_edge_body,
        out_type=[
            jax.ShapeDtypeStruct((_NPAD, 512), jnp.float32),
            jax.ShapeDtypeStruct((_NPAD,), jnp.float32),
        ],
        mesh=mesh,
        compiler_params=pltpu.CompilerParams(needs_layout_passes=False),
        scratch_types=[
            pltpu.VMEM((80,), jnp.int32),
            pltpu.VMEM((_KB,), jnp.int32),
            pltpu.VMEM((_KB + 16,), jnp.int32),
            pltpu.VMEM((_KB,), jnp.int32),
            pltpu.VMEM((_K, 128), jnp.float32),
            pltpu.VMEM((_K, 128), jnp.float32),
            pltpu.VMEM((_K, 128), jnp.float32),
            pltpu.VMEM((_K,), jnp.int32),
            pltpu.VMEM((_K, 128), jnp.float32),
            pltpu.VMEM((_K, 128), jnp.float32),
            pltpu.VMEM((_K, 128), jnp.float32),
            pltpu.VMEM((_K,), jnp.int32),
            pltpu.VMEM((_C + 1, 512), jnp.float32),
            pltpu.VMEM((_C + 16,), jnp.float32),
            pltpu.SemaphoreType.DMA,
            pltpu.SemaphoreType.DMA,
            pltpu.SemaphoreType.DMA,
            pltpu.SemaphoreType.DMA,
            pltpu.SemaphoreType.DMA,
            pltpu.SemaphoreType.DMA,
        ],
    )(a, bmat, ce_l, ssrc_p, sdst_p, perm_p, bounds)


def kernel(h, e, snorm_n, snorm_e, params, edge_index):
    src = edge_index[0]
    dst = edge_index[1]
    n = h.shape[0]

    perm = jnp.argsort(dst).astype(jnp.int32)
    sdst = dst[perm]
    ssrc = src[perm]
    big = jnp.full((_KB,), _NPAD, jnp.int32)
    sdst_p = jnp.concatenate([sdst, big])
    ssrc_p = jnp.concatenate([ssrc, jnp.zeros((_KB,), jnp.int32)])
    perm_p = jnp.concatenate([perm, jnp.zeros((_KB,), jnp.int32)])
    chunk_starts = jnp.arange(0, _NPAD + 1, _C, dtype=jnp.int32)
    bounds = jnp.searchsorted(sdst, chunk_starts).astype(jnp.int32)
    bounds = jnp.concatenate(
        [bounds, jnp.zeros((80 - bounds.shape[0],), jnp.int32)])

    ce_l = [
        _mm(e, params['layers'][l]['pre_W'][256:, :],
            params['layers'][l]['pre_b'])
        for l in range(_N_LAYERS)
    ]

    h = _mm(h, params['emb_W'], params['emb_b'])

    for l in range(_N_LAYERS):
        p = params['layers'][l]
        h_in = h
        a = _mm(h, p['pre_W'][:128, :], jnp.zeros((128,), jnp.float32))
        bmat = _mm(h, p['pre_W'][128:256, :], jnp.zeros((128,), jnp.float32))
        sc_out, deg = _sc_edge(a, bmat, ce_l[l], ssrc_p, sdst_p, perm_p,
                               bounds)

        wh = p['post_W'][:128, :]
        wabc = jnp.concatenate(
            [p['post_W'][128:640, :], p['post_W'][640:1152, :],
             p['post_W'][1152:1664, :]], axis=1)
        hp, stats = _post(h, sc_out, deg.reshape(_NPAD, 1), wh, wabc,
                          p['post_b'], snorm_n)
        h = _bn_res(h_in, hp, stats, p['bn_g'], p['bn_b'])

    return _readout(h, params['read'])


# speedup vs baseline: 4.2071x; 1.0647x over previous
import functools

import jax
import jax.numpy as jnp
import numpy as np
from jax import lax
from jax.experimental import pallas as pl
from jax.experimental.pallas import tpu as pltpu
from jax.experimental.pallas import tpu_sc as plsc

_EPS = 1e-5
_AVG_D_LOG = float(np.log(17.0))
_N_LAYERS = 4

_K = 48
_NSUB = 16
_KB = _K * _NSUB
_C = 160
_NCHUNK = 64
_NPAD = _C * _NCHUNK


def _mm_body(x_ref, w_ref, b_ref, o_ref):
    o_ref[...] = (
        jnp.dot(x_ref[...], w_ref[...], preferred_element_type=jnp.float32)
        + b_ref[...]
    )


def _mm(x, w, b, bm=1000):
    m, k = x.shape
    n = w.shape[1]
    assert m % bm == 0, (m, bm)
    return pl.pallas_call(
        _mm_body,
        grid=(m // bm,),
        in_specs=[
            pl.BlockSpec((bm, k), lambda i: (i, 0)),
            pl.BlockSpec((k, n), lambda i: (0, 0)),
            pl.BlockSpec((1, n), lambda i: (0, 0)),
        ],
        out_specs=pl.BlockSpec((bm, n), lambda i: (i, 0)),
        out_shape=jax.ShapeDtypeStruct((m, n), jnp.float32),
    )(x, w, b.reshape(1, n))


def _post_body(h_ref, sc_ref, deg_ref, wh_ref, wabc_ref, b_ref,
               sn_ref, o_ref, stat_ref):
    sc = sc_ref[...]
    deg = deg_ref[...]
    degc = jnp.maximum(deg, 1.0)
    mask = deg > 0.0
    mean = sc[:, 0:128] / degc
    var = sc[:, 128:256] / degc - mean * mean
    std = jnp.sqrt(jnp.maximum(var, 0.0) + _EPS)
    mx = jnp.where(mask, sc[:, 256:384], 0.0)
    mn = jnp.where(mask, sc[:, 384:512], 0.0)
    agg = jnp.concatenate([mean, mx, mn, std], axis=1)
    logd = jnp.log(deg + 1.0)
    amp = logd * (1.0 / _AVG_D_LOG)
    att = _AVG_D_LOG / jnp.maximum(logd, _EPS)
    u = jnp.dot(agg, wabc_ref[...], preferred_element_type=jnp.float32)
    hp = (
        jnp.dot(h_ref[...], wh_ref[...], preferred_element_type=jnp.float32)
        + u[:, :128]
        + amp * u[:, 128:256]
        + att * u[:, 256:384]
        + b_ref[...]
    )
    hp = hp * sn_ref[...]
    o_ref[...] = hp

    @pl.when(pl.program_id(0) == 0)
    def _():
        stat_ref[...] = jnp.zeros_like(stat_ref)

    stat_ref[0, :] += jnp.sum(hp, axis=0)
    stat_ref[1, :] += jnp.sum(hp * hp, axis=0)


def _post(h, sc, deg, wh, wabc, b, sn, bm=1000):
    m = h.shape[0]
    assert m % bm == 0
    return pl.pallas_call(
        _post_body,
        grid=(m // bm,),
        in_specs=[
            pl.BlockSpec((bm, 128), lambda i: (i, 0)),
            pl.BlockSpec((bm, 512), lambda i: (i, 0)),
            pl.BlockSpec((bm, 1), lambda i: (i, 0)),
            pl.BlockSpec((128, 128), lambda i: (0, 0)),
            pl.BlockSpec((512, 384), lambda i: (0, 0)),
            pl.BlockSpec((1, 128), lambda i: (0, 0)),
            pl.BlockSpec((bm, 1), lambda i: (i, 0)),
        ],
        out_specs=[
            pl.BlockSpec((bm, 128), lambda i: (i, 0)),
            pl.BlockSpec((2, 128), lambda i: (0, 0)),
        ],
        out_shape=[
            jax.ShapeDtypeStruct((m, 128), jnp.float32),
            jax.ShapeDtypeStruct((2, 128), jnp.float32),
        ],
    )(h, sc, deg, wh, wabc, b.reshape(1, 128), sn)


def _bn_res_body(hin_ref, hp_ref, stat_ref, g_ref, bb_ref, o_ref):
    n = hin_ref.shape[0] * pl.num_programs(0)
    mu = stat_ref[0:1, :] * (1.0 / n)
    var = stat_ref[1:2, :] * (1.0 / n) - mu * mu
    rs = lax.rsqrt(var + _EPS)
    hp = (hp_ref[...] - mu) * rs * g_ref[...] + bb_ref[...]
    o_ref[...] = hin_ref[...] + jnp.maximum(hp, 0.0)


def _bn_res(h_in, hp, stats, g, bb, bm=1000):
    m = h_in.shape[0]
    row = lambda a: a.reshape(1, 128)
    return pl.pallas_call(
        _bn_res_body,
        grid=(m // bm,),
        in_specs=[
            pl.BlockSpec((bm, 128), lambda i: (i, 0)),
            pl.BlockSpec((bm, 128), lambda i: (i, 0)),
            pl.BlockSpec((2, 128), lambda i: (0, 0)),
            pl.BlockSpec((1, 128), lambda i: (0, 0)),
            pl.BlockSpec((1, 128), lambda i: (0, 0)),
        ],
        out_specs=pl.BlockSpec((bm, 128), lambda i: (i, 0)),
        out_shape=jax.ShapeDtypeStruct((m, 128), jnp.float32),
    )(h_in, hp, stats, row(g), row(bb))


def _readout_body(h_ref, w1_ref, b1_ref, w2_ref, b2_ref, w3_ref, b3_ref,
                  o_ref, acc_ref):
    @pl.when(pl.program_id(0) == 0)
    def _():
        acc_ref[...] = jnp.zeros_like(acc_ref)

    acc_ref[0, :] += jnp.sum(h_ref[...], axis=0)

    @pl.when(pl.program_id(0) == pl.num_programs(0) - 1)
    def _():
        hg = acc_ref[...] * (1.0 / h_ref.shape[0] / pl.num_programs(0))
        x = jnp.maximum(jnp.dot(hg, w1_ref[...],
                                preferred_element_type=jnp.float32)
                        + b1_ref[...], 0.0)
        x = jnp.maximum(jnp.dot(x, w2_ref[...],
                                preferred_element_type=jnp.float32)
                        + b2_ref[...], 0.0)
        o_ref[...] = (jnp.dot(x, w3_ref[...],
                              preferred_element_type=jnp.float32)
                      + b3_ref[...])


def _readout(h, r, bm=1000):
    m = h.shape[0]
    return pl.pallas_call(
        _readout_body,
        grid=(m // bm,),
        in_specs=[
            pl.BlockSpec((bm, 128), lambda i: (i, 0)),
            pl.BlockSpec((128, 64), lambda i: (0, 0)),
            pl.BlockSpec((1, 64), lambda i: (0, 0)),
            pl.BlockSpec((64, 32), lambda i: (0, 0)),
            pl.BlockSpec((1, 32), lambda i: (0, 0)),
            pl.BlockSpec((32, 64), lambda i: (0, 0)),
            pl.BlockSpec((1, 64), lambda i: (0, 0)),
        ],
        out_specs=[
            pl.BlockSpec((1, 64), lambda i: (0, 0)),
            pl.BlockSpec((1, 128), lambda i: (0, 0)),
        ],
        out_shape=[
            jax.ShapeDtypeStruct((1, 64), jnp.float32),
            jax.ShapeDtypeStruct((1, 128), jnp.float32),
        ],
    )(h, r['W1'], r['b1'].reshape(1, 64), r['W2'], r['b2'].reshape(1, 32),
      r['W3'], r['b3'].reshape(1, 64))[0]


def _sc_edge_body(a_hbm, b_hbm, ce_hbm, ssrc_hbm, sdst_hbm, perm_hbm,
                  bounds_hbm, out_hbm, deg_hbm,
                  bounds_v, sidx, didx, pidx,
                  bufa0, bufb0, bufc0, didxg0,
                  bufa1, bufb1, bufc1, didxg1, acc, dacc,
                  sem_a0, sem_b0, sem_c0, sem_a1, sem_b1, sem_c1):
    wid = lax.axis_index("s") * 2 + lax.axis_index("c")
    pltpu.sync_copy(bounds_hbm, bounds_v)
    zero16 = jnp.zeros((16,), jnp.float32)
    neg16 = jnp.full((16,), -3e38, jnp.float32)
    pos16 = jnp.full((16,), 3e38, jnp.float32)

    for sub in range(2):
        chunk = wid * 2 + sub
        clo = chunk * _C

        def init_row(i, _):
            for c in range(16):
                acc[i, pl.ds(c * 16, 16)] = zero16
            for c in range(8):
                acc[i, pl.ds(256 + c * 16, 16)] = neg16
                acc[i, pl.ds(384 + c * 16, 16)] = pos16
            return 0

        lax.fori_loop(0, _C + 1, init_row, 0)

        def init_deg(i, _):
            dacc[pl.ds(i * 16, 16)] = zero16
            return 0

        lax.fori_loop(0, (_C + 16) // 16, init_deg, 0)

        bv = bounds_v[pl.ds(chunk, 16)]
        elo = bv[0]
        ehi = bv[1]
        elo8 = (elo // 8) * 8

        nmax = a_hbm.shape[0] - 1
        bufs = ((bufa0, bufb0, bufc0, didxg0, sem_a0, sem_b0, sem_c0),
                (bufa1, bufb1, bufc1, didxg1, sem_a1, sem_b1, sem_c1))
        ones16 = jnp.ones((16,), jnp.float32)

        def issue(s, par):
            ba, bb, bc, dg, sa, sb, sc = bufs[par]
            off = s * _K
            for g in range(_K // 16):
                dg[pl.ds(g * 16, 16)] = jnp.minimum(
                    didx[pl.ds(off + g * 16, 16)], nmax)
            pltpu.async_copy(a_hbm.at[sidx.at[pl.ds(off, _K)]], ba, sa)
            pltpu.async_copy(b_hbm.at[dg], bb, sb)
            pltpu.async_copy(ce_hbm.at[pidx.at[pl.ds(off, _K)]], bc, sc)

        def compute(s, par):
            ba, bb, bc, dg, sa, sb, sc = bufs[par]
            pltpu.make_async_copy(a_hbm.at[sidx.at[pl.ds(0, _K)]], ba,
                                  sa).wait()
            pltpu.make_async_copy(b_hbm.at[dg], bb, sb).wait()
            pltpu.make_async_copy(ce_hbm.at[pidx.at[pl.ds(0, _K)]], bc,
                                  sc).wait()
            base = s * _K
            for g in range(_K // 16):
                dv = didx[pl.ds(base + g * 16, 16)] - clo
                ok = (dv >= 0) & (dv < _C)
                dvc = jnp.where(ok, dv, _C)
                plsc.addupdate_scatter(dacc, [dvc], ones16, mask=ok)

            def edge(j4, _):
                djs = []
                for u in range(4):
                    j = 4 * j4 + u
                    d = didx[pl.ds(base + j, 16)][0]
                    dj = d - clo
                    inr = (dj >= 0) & (dj < _C)
                    djs.append((j, jnp.where(inr, dj, _C)))
                for c in range(8):
                    off = c * 16
                    for j, dj in djs:
                        m = jnp.maximum(
                            ba[j, pl.ds(off, 16)]
                            + bb[j, pl.ds(off, 16)]
                            + bc[j, pl.ds(off, 16)], 0.0)
                        plsc.addupdate(acc.at[dj, pl.ds(off, 16)], m)
                        plsc.addupdate(acc.at[dj, pl.ds(128 + off, 16)],
                                       m * m)
                        acc[dj, pl.ds(256 + off, 16)] = jnp.maximum(
                            acc[dj, pl.ds(256 + off, 16)], m)
                        acc[dj, pl.ds(384 + off, 16)] = jnp.minimum(
                            acc[dj, pl.ds(384 + off, 16)], m)
                return 0

            lax.fori_loop(0, _K // 4, edge, 0)

        nbig = (ehi - elo8 + _KB - 1) // _KB
        nsub_total = (ehi - elo8 + _K - 1) // _K

        def big_batch(t, _):
            base = elo8 + t * _KB
            pltpu.sync_copy(ssrc_hbm.at[pl.ds(base, _KB)], sidx)
            pltpu.sync_copy(sdst_hbm.at[pl.ds(base, _KB)],
                            didx.at[pl.ds(0, _KB)])
            pltpu.sync_copy(perm_hbm.at[pl.ds(base, _KB)], pidx)
            ns = jnp.minimum(nsub_total - t * _NSUB, _NSUB)
            issue(0, 0)

            def pair(i, _):
                s0 = 2 * i

                @pl.when(s0 + 1 < ns)
                def _():
                    issue(s0 + 1, 1)

                compute(s0, 0)
                s1 = s0 + 1

                @pl.when(s1 < ns)
                def _():
                    @pl.when(s1 + 1 < ns)
                    def _():
                        issue(s1 + 1, 0)

                    compute(s1, 1)

                return 0

            lax.fori_loop(0, (ns + 1) // 2, pair, 0)
            return 0

        lax.fori_loop(0, nbig, big_batch, 0)

        pltpu.sync_copy(acc.at[pl.ds(0, _C)], out_hbm.at[pl.ds(clo, _C)])
        pltpu.sync_copy(dacc.at[pl.ds(0, _C)], deg_hbm.at[pl.ds(clo, _C)])


@functools.partial(jax.jit, static_argnums=())
def _sc_edge(a, bmat, ce_l, ssrc_p, sdst_p, perm_p, bounds):
    mesh = plsc.VectorSubcoreMesh(core_axis_name="c", subcore_axis_name="s",
                                  num_cores=2, num_subcores=16)
    return pl.kernel(
        _sc_edge_body,
        out_type=[
            jax.ShapeDtypeStruct((_NPAD, 512), jnp.float32),
            jax.ShapeDtypeStruct((_NPAD,), jnp.float32),
        ],
        mesh=mesh,
        compiler_params=pltpu.CompilerParams(needs_layout_passes=False),
        scratch_types=[
            pltpu.VMEM((80,), jnp.int32),
            pltpu.VMEM((_KB,), jnp.int32),
            pltpu.VMEM((_KB + 16,), jnp.int32),
            pltpu.VMEM((_KB,), jnp.int32),
            pltpu.VMEM((_K, 128), jnp.float32),
            pltpu.VMEM((_K, 128), jnp.float32),
            pltpu.VMEM((_K, 128), jnp.float32),
            pltpu.VMEM((_K,), jnp.int32),
            pltpu.VMEM((_K, 128), jnp.float32),
            pltpu.VMEM((_K, 128), jnp.float32),
            pltpu.VMEM((_K, 128), jnp.float32),
            pltpu.VMEM((_K,), jnp.int32),
            pltpu.VMEM((_C + 1, 512), jnp.float32),
            pltpu.VMEM((_C + 16,), jnp.float32),
            pltpu.SemaphoreType.DMA,
            pltpu.SemaphoreType.DMA,
            pltpu.SemaphoreType.DMA,
            pltpu.SemaphoreType.DMA,
            pltpu.SemaphoreType.DMA,
            pltpu.SemaphoreType.DMA,
        ],
    )(a, bmat, ce_l, ssrc_p, sdst_p, perm_p, bounds)


def kernel(h, e, snorm_n, snorm_e, params, edge_index):
    src = edge_index[0]
    dst = edge_index[1]
    n = h.shape[0]

    perm = jnp.argsort(dst).astype(jnp.int32)
    sdst = dst[perm]
    ssrc = src[perm]
    big = jnp.full((_KB,), _NPAD, jnp.int32)
    sdst_p = jnp.concatenate([sdst, big])
    ssrc_p = jnp.concatenate([ssrc, jnp.zeros((_KB,), jnp.int32)])
    perm_p = jnp.concatenate([perm, jnp.zeros((_KB,), jnp.int32)])
    chunk_starts = jnp.arange(0, _NPAD + 1, _C, dtype=jnp.int32)
    bounds = jnp.searchsorted(sdst, chunk_starts).astype(jnp.int32)
    bounds = jnp.concatenate(
        [bounds, jnp.zeros((80 - bounds.shape[0],), jnp.int32)])

    ce_l = [
        _mm(e, params['layers'][l]['pre_W'][256:, :],
            params['layers'][l]['pre_b'])
        for l in range(_N_LAYERS)
    ]

    h = _mm(h, params['emb_W'], params['emb_b'])

    for l in range(_N_LAYERS):
        p = params['layers'][l]
        h_in = h
        a = _mm(h, p['pre_W'][:128, :], jnp.zeros((128,), jnp.float32))
        bmat = _mm(h, p['pre_W'][128:256, :], jnp.zeros((128,), jnp.float32))
        sc_out, deg = _sc_edge(a, bmat, ce_l[l], ssrc_p, sdst_p, perm_p,
                               bounds)

        wh = p['post_W'][:128, :]
        wabc = jnp.concatenate(
            [p['post_W'][128:640, :], p['post_W'][640:1152, :],
             p['post_W'][1152:1664, :]], axis=1)
        hp, stats = _post(h, sc_out, deg.reshape(_NPAD, 1), wh, wabc,
                          p['post_b'], snorm_n)
        h = _bn_res(h_in, hp, stats, p['bn_g'], p['bn_b'])

    return _readout(h, params['read'])


# speedup vs baseline: 4.2251x; 1.0043x over previous
import functools

import jax
import jax.numpy as jnp
import numpy as np
from jax import lax
from jax.experimental import pallas as pl
from jax.experimental.pallas import tpu as pltpu
from jax.experimental.pallas import tpu_sc as plsc

_EPS = 1e-5
_AVG_D_LOG = float(np.log(17.0))
_N_LAYERS = 4

_K = 32
_NSUB = 16
_KB = _K * _NSUB
_C = 160
_NCHUNK = 64
_NPAD = _C * _NCHUNK


def _mm_body(x_ref, w_ref, b_ref, o_ref):
    o_ref[...] = (
        jnp.dot(x_ref[...], w_ref[...], preferred_element_type=jnp.float32)
        + b_ref[...]
    )


def _mm(x, w, b, bm=1000):
    m, k = x.shape
    n = w.shape[1]
    assert m % bm == 0, (m, bm)
    return pl.pallas_call(
        _mm_body,
        grid=(m // bm,),
        in_specs=[
            pl.BlockSpec((bm, k), lambda i: (i, 0)),
            pl.BlockSpec((k, n), lambda i: (0, 0)),
            pl.BlockSpec((1, n), lambda i: (0, 0)),
        ],
        out_specs=pl.BlockSpec((bm, n), lambda i: (i, 0)),
        out_shape=jax.ShapeDtypeStruct((m, n), jnp.float32),
    )(x, w, b.reshape(1, n))


def _post_body(h_ref, sc_ref, deg_ref, wh_ref, wabc_ref, b_ref,
               sn_ref, o_ref, stat_ref):
    sc = sc_ref[...]
    deg = deg_ref[...]
    degc = jnp.maximum(deg, 1.0)
    mask = deg > 0.0
    mean = sc[:, 0:128] / degc
    var = sc[:, 128:256] / degc - mean * mean
    std = jnp.sqrt(jnp.maximum(var, 0.0) + _EPS)
    mx = jnp.where(mask, sc[:, 256:384], 0.0)
    mn = jnp.where(mask, sc[:, 384:512], 0.0)
    agg = jnp.concatenate([mean, mx, mn, std], axis=1)
    logd = jnp.log(deg + 1.0)
    amp = logd * (1.0 / _AVG_D_LOG)
    att = _AVG_D_LOG / jnp.maximum(logd, _EPS)
    u = jnp.dot(agg, wabc_ref[...], preferred_element_type=jnp.float32)
    hp = (
        jnp.dot(h_ref[...], wh_ref[...], preferred_element_type=jnp.float32)
        + u[:, :128]
        + amp * u[:, 128:256]
        + att * u[:, 256:384]
        + b_ref[...]
    )
    hp = hp * sn_ref[...]
    o_ref[...] = hp

    @pl.when(pl.program_id(0) == 0)
    def _():
        stat_ref[...] = jnp.zeros_like(stat_ref)

    stat_ref[0, :] += jnp.sum(hp, axis=0)
    stat_ref[1, :] += jnp.sum(hp * hp, axis=0)


def _post(h, sc, deg, wh, wabc, b, sn, bm=1000):
    m = h.shape[0]
    assert m % bm == 0
    return pl.pallas_call(
        _post_body,
        grid=(m // bm,),
        in_specs=[
            pl.BlockSpec((bm, 128), lambda i: (i, 0)),
            pl.BlockSpec((bm, 512), lambda i: (i, 0)),
            pl.BlockSpec((bm, 1), lambda i: (i, 0)),
            pl.BlockSpec((128, 128), lambda i: (0, 0)),
            pl.BlockSpec((512, 384), lambda i: (0, 0)),
            pl.BlockSpec((1, 128), lambda i: (0, 0)),
            pl.BlockSpec((bm, 1), lambda i: (i, 0)),
        ],
        out_specs=[
            pl.BlockSpec((bm, 128), lambda i: (i, 0)),
            pl.BlockSpec((2, 128), lambda i: (0, 0)),
        ],
        out_shape=[
            jax.ShapeDtypeStruct((m, 128), jnp.float32),
            jax.ShapeDtypeStruct((2, 128), jnp.float32),
        ],
    )(h, sc, deg, wh, wabc, b.reshape(1, 128), sn)


def _bn_res_body(hin_ref, hp_ref, stat_ref, g_ref, bb_ref, o_ref):
    n = hin_ref.shape[0] * pl.num_programs(0)
    mu = stat_ref[0:1, :] * (1.0 / n)
    var = stat_ref[1:2, :] * (1.0 / n) - mu * mu
    rs = lax.rsqrt(var + _EPS)
    hp = (hp_ref[...] - mu) * rs * g_ref[...] + bb_ref[...]
    o_ref[...] = hin_ref[...] + jnp.maximum(hp, 0.0)


def _bn_res(h_in, hp, stats, g, bb, bm=1000):
    m = h_in.shape[0]
    row = lambda a: a.reshape(1, 128)
    return pl.pallas_call(
        _bn_res_body,
        grid=(m // bm,),
        in_specs=[
            pl.BlockSpec((bm, 128), lambda i: (i, 0)),
            pl.BlockSpec((bm, 128), lambda i: (i, 0)),
            pl.BlockSpec((2, 128), lambda i: (0, 0)),
            pl.BlockSpec((1, 128), lambda i: (0, 0)),
            pl.BlockSpec((1, 128), lambda i: (0, 0)),
        ],
        out_specs=pl.BlockSpec((bm, 128), lambda i: (i, 0)),
        out_shape=jax.ShapeDtypeStruct((m, 128), jnp.float32),
    )(h_in, hp, stats, row(g), row(bb))


def _readout_body(h_ref, w1_ref, b1_ref, w2_ref, b2_ref, w3_ref, b3_ref,
                  o_ref, acc_ref):
    @pl.when(pl.program_id(0) == 0)
    def _():
        acc_ref[...] = jnp.zeros_like(acc_ref)

    acc_ref[0, :] += jnp.sum(h_ref[...], axis=0)

    @pl.when(pl.program_id(0) == pl.num_programs(0) - 1)
    def _():
        hg = acc_ref[...] * (1.0 / h_ref.shape[0] / pl.num_programs(0))
        x = jnp.maximum(jnp.dot(hg, w1_ref[...],
                                preferred_element_type=jnp.float32)
                        + b1_ref[...], 0.0)
        x = jnp.maximum(jnp.dot(x, w2_ref[...],
                                preferred_element_type=jnp.float32)
                        + b2_ref[...], 0.0)
        o_ref[...] = (jnp.dot(x, w3_ref[...],
                              preferred_element_type=jnp.float32)
                      + b3_ref[...])


def _readout(h, r, bm=1000):
    m = h.shape[0]
    return pl.pallas_call(
        _readout_body,
        grid=(m // bm,),
        in_specs=[
            pl.BlockSpec((bm, 128), lambda i: (i, 0)),
            pl.BlockSpec((128, 64), lambda i: (0, 0)),
            pl.BlockSpec((1, 64), lambda i: (0, 0)),
            pl.BlockSpec((64, 32), lambda i: (0, 0)),
            pl.BlockSpec((1, 32), lambda i: (0, 0)),
            pl.BlockSpec((32, 64), lambda i: (0, 0)),
            pl.BlockSpec((1, 64), lambda i: (0, 0)),
        ],
        out_specs=[
            pl.BlockSpec((1, 64), lambda i: (0, 0)),
            pl.BlockSpec((1, 128), lambda i: (0, 0)),
        ],
        out_shape=[
            jax.ShapeDtypeStruct((1, 64), jnp.float32),
            jax.ShapeDtypeStruct((1, 128), jnp.float32),
        ],
    )(h, r['W1'], r['b1'].reshape(1, 64), r['W2'], r['b2'].reshape(1, 32),
      r['W3'], r['b3'].reshape(1, 64))[0]


def _sc_edge_body(a_hbm, b_hbm, ce_hbm, ssrc_hbm, sdst_hbm, perm_hbm,
                  bounds_hbm, out_hbm, deg_hbm,
                  bounds_v, sidx, didx, pidx,
                  bufa0, bufc0, bufa1, bufc1, bufn, acc, dacc,
                  sem_a0, sem_c0, sem_a1, sem_c1):
    wid = lax.axis_index("s") * 2 + lax.axis_index("c")
    pltpu.sync_copy(bounds_hbm, bounds_v)
    zero16 = jnp.zeros((16,), jnp.float32)
    neg16 = jnp.full((16,), -3e38, jnp.float32)
    pos16 = jnp.full((16,), 3e38, jnp.float32)

    for sub in range(2):
        chunk = wid * 2 + sub
        clo = chunk * _C

        def init_row(i, _):
            for c in range(16):
                acc[i, pl.ds(c * 16, 16)] = zero16
            for c in range(8):
                acc[i, pl.ds(256 + c * 16, 16)] = neg16
                acc[i, pl.ds(384 + c * 16, 16)] = pos16
            return 0

        lax.fori_loop(0, _C + 1, init_row, 0)

        def init_deg(i, _):
            dacc[pl.ds(i * 16, 16)] = zero16
            return 0

        lax.fori_loop(0, (_C + 16) // 16, init_deg, 0)

        bv = bounds_v[pl.ds(chunk, 16)]
        elo = bv[0]
        ehi = bv[1]
        elo8 = (elo // 8) * 8

        pltpu.sync_copy(b_hbm.at[pl.ds(clo, _C)], bufn.at[pl.ds(0, _C)])
        for c in range(8):
            bufn[_C, pl.ds(c * 16, 16)] = zero16

        bufs = ((bufa0, bufc0, sem_a0, sem_c0),
                (bufa1, bufc1, sem_a1, sem_c1))
        ones16 = jnp.ones((16,), jnp.float32)

        def issue(s, par):
            ba, bc, sa, sc = bufs[par]
            off = s * _K
            pltpu.async_copy(a_hbm.at[sidx.at[pl.ds(off, _K)]], ba, sa)
            pltpu.async_copy(ce_hbm.at[pidx.at[pl.ds(off, _K)]], bc, sc)

        def compute(s, par):
            ba, bc, sa, sc = bufs[par]
            pltpu.make_async_copy(a_hbm.at[sidx.at[pl.ds(0, _K)]], ba,
                                  sa).wait()
            pltpu.make_async_copy(ce_hbm.at[pidx.at[pl.ds(0, _K)]], bc,
                                  sc).wait()
            base = s * _K
            for g in range(_K // 16):
                dv = didx[pl.ds(base + g * 16, 16)] - clo
                ok = (dv >= 0) & (dv < _C)
                dvc = jnp.where(ok, dv, _C)
                plsc.addupdate_scatter(dacc, [dvc], ones16, mask=ok)

            def edge(j4, _):
                djs = []
                for u in range(4):
                    j = 4 * j4 + u
                    d = didx[pl.ds(base + j, 16)][0]
                    dj = d - clo
                    inr = (dj >= 0) & (dj < _C)
                    djs.append((j, jnp.where(inr, dj, _C)))
                for c in range(8):
                    off = c * 16
                    for j, dj in djs:
                        m = jnp.maximum(
                            ba[j, pl.ds(off, 16)]
                            + bufn[dj, pl.ds(off, 16)]
                            + bc[j, pl.ds(off, 16)], 0.0)
                        plsc.addupdate(acc.at[dj, pl.ds(off, 16)], m)
                        plsc.addupdate(acc.at[dj, pl.ds(128 + off, 16)],
                                       m * m)
                        acc[dj, pl.ds(256 + off, 16)] = jnp.maximum(
                            acc[dj, pl.ds(256 + off, 16)], m)
                        acc[dj, pl.ds(384 + off, 16)] = jnp.minimum(
                            acc[dj, pl.ds(384 + off, 16)], m)
                return 0

            lax.fori_loop(0, _K // 4, edge, 0)

        nbig = (ehi - elo8 + _KB - 1) // _KB
        nsub_total = (ehi - elo8 + _K - 1) // _K

        def big_batch(t, _):
            base = elo8 + t * _KB
            pltpu.sync_copy(ssrc_hbm.at[pl.ds(base, _KB)], sidx)
            pltpu.sync_copy(sdst_hbm.at[pl.ds(base, _KB)],
                            didx.at[pl.ds(0, _KB)])
            pltpu.sync_copy(perm_hbm.at[pl.ds(base, _KB)], pidx)
            ns = jnp.minimum(nsub_total - t * _NSUB, _NSUB)
            issue(0, 0)

            def pair(i, _):
                s0 = 2 * i

                @pl.when(s0 + 1 < ns)
                def _():
                    issue(s0 + 1, 1)

                compute(s0, 0)
                s1 = s0 + 1

                @pl.when(s1 < ns)
                def _():
                    @pl.when(s1 + 1 < ns)
                    def _():
                        issue(s1 + 1, 0)

                    compute(s1, 1)

                return 0

            lax.fori_loop(0, (ns + 1) // 2, pair, 0)
            return 0

        lax.fori_loop(0, nbig, big_batch, 0)

        pltpu.sync_copy(acc.at[pl.ds(0, _C)], out_hbm.at[pl.ds(clo, _C)])
        pltpu.sync_copy(dacc.at[pl.ds(0, _C)], deg_hbm.at[pl.ds(clo, _C)])


@functools.partial(jax.jit, static_argnums=())
def _sc_edge(a, bmat, ce_l, ssrc_p, sdst_p, perm_p, bounds):
    mesh = plsc.VectorSubcoreMesh(core_axis_name="c", subcore_axis_name="s",
                                  num_cores=2, num_subcores=16)
    return pl.kernel(
        _sc_edge_body,
        out_type=[
            jax.ShapeDtypeStruct((_NPAD, 512), jnp.float32),
            jax.ShapeDtypeStruct((_NPAD,), jnp.float32),
        ],
        mesh=mesh,
        compiler_params=pltpu.CompilerParams(needs_layout_passes=False),
        scratch_types=[
            pltpu.VMEM((80,), jnp.int32),
            pltpu.VMEM((_KB,), jnp.int32),
            pltpu.VMEM((_KB + 16,), jnp.int32),
            pltpu.VMEM((_KB,), jnp.int32),
            pltpu.VMEM((_K, 128), jnp.float32),
            pltpu.VMEM((_K, 128), jnp.float32),
            pltpu.VMEM((_K, 128), jnp.float32),
            pltpu.VMEM((_K, 128), jnp.float32),
            pltpu.VMEM((_C + 1, 128), jnp.float32),
            pltpu.VMEM((_C + 1, 512), jnp.float32),
            pltpu.VMEM((_C + 16,), jnp.float32),
            pltpu.SemaphoreType.DMA,
            pltpu.SemaphoreType.DMA,
            pltpu.SemaphoreType.DMA,
            pltpu.SemaphoreType.DMA,
        ],
    )(a, bmat, ce_l, ssrc_p, sdst_p, perm_p, bounds)


def kernel(h, e, snorm_n, snorm_e, params, edge_index):
    src = edge_index[0]
    dst = edge_index[1]
    n = h.shape[0]

    perm = jnp.argsort(dst).astype(jnp.int32)
    sdst = dst[perm]
    ssrc = src[perm]
    big = jnp.full((_KB,), _NPAD, jnp.int32)
    sdst_p = jnp.concatenate([sdst, big])
    ssrc_p = jnp.concatenate([ssrc, jnp.zeros((_KB,), jnp.int32)])
    perm_p = jnp.concatenate([perm, jnp.zeros((_KB,), jnp.int32)])
    chunk_starts = jnp.arange(0, _NPAD + 1, _C, dtype=jnp.int32)
    bounds = jnp.searchsorted(sdst, chunk_starts).astype(jnp.int32)
    bounds = jnp.concatenate(
        [bounds, jnp.zeros((80 - bounds.shape[0],), jnp.int32)])

    ce_l = [
        _mm(e, params['layers'][l]['pre_W'][256:, :],
            params['layers'][l]['pre_b'])
        for l in range(_N_LAYERS)
    ]

    h = _mm(h, params['emb_W'], params['emb_b'])

    for l in range(_N_LAYERS):
        p = params['layers'][l]
        h_in = h
        a = _mm(h, p['pre_W'][:128, :], jnp.zeros((128,), jnp.float32))
        bmat = _mm(h, p['pre_W'][128:256, :], jnp.zeros((128,), jnp.float32))
        bmat = jnp.concatenate(
            [bmat, jnp.zeros((_NPAD - n, 128), jnp.float32)])
        sc_out, deg = _sc_edge(a, bmat, ce_l[l], ssrc_p, sdst_p, perm_p,
                               bounds)

        wh = p['post_W'][:128, :]
        wabc = jnp.concatenate(
            [p['post_W'][128:640, :], p['post_W'][640:1152, :],
             p['post_W'][1152:1664, :]], axis=1)
        hp, stats = _post(h, sc_out, deg.reshape(_NPAD, 1), wh, wabc,
                          p['post_b'], snorm_n)
        h = _bn_res(h_in, hp, stats, p['bn_g'], p['bn_b'])

    return _readout(h, params['read'])
